# baseline (speedup 1.0000x reference)
import jax
import jax.numpy as jnp
from jax.experimental import pallas as pl
from jax.experimental.pallas import tpu as pltpu

_N = 1048576
_W = 128
_ROWS = _N // _W
_K = 2048
_G = _ROWS // _K
_Q = 128


def _fold(x):
    return jnp.sum(x.reshape(-1, 8, _W), axis=0)


def _body(gt_ref, p_ref, bt_ref, bp_ref, tr_ref, pr_ref, out_ref,
          acc_ref, s_ref):
    i = pl.program_id(0)

    @pl.when(i == 0)
    def _init():
        acc_ref[...] = jnp.zeros_like(acc_ref)
        ri = jax.lax.broadcasted_iota(jnp.int32, (4 * _Q, _Q), 0)
        ki = jax.lax.broadcasted_iota(jnp.int32, (4 * _Q, _Q), 1)
        s_ref[...] = jnp.where(ri // 4 == ki, 1.0, 0.0).astype(jnp.bfloat16)

    gt = gt_ref[...]
    p = p_ref[...]
    tr = tr_ref[...]
    pr = pr_ref[...]

    one = jnp.float32(1.0)
    zero = jnp.float32(0.0)

    mask_f = gt >= 0.0
    mf = jnp.where(mask_f, one, zero)
    q = jnp.where(gt == 1.0, p, one - p)
    bce_f = mf * jnp.log(q)

    mb = jnp.abs(gt)
    mr = jnp.where(tr >= 0.0, mb, zero)
    qr = jnp.where(tr == 1.0, pr, one - pr)
    bce_r = mr * jnp.log(qr)

    d = bp_ref[...] - bt_ref[...]
    sq = d * d
    mb_bf = mb.astype(jnp.bfloat16)
    sqe = jnp.zeros((8, _W), jnp.float32)
    for j in range(_K // _Q):
        e_j = jax.lax.dot(s_ref[...], mb_bf[_Q * j:_Q * (j + 1), :],
                          preferred_element_type=jnp.float32)
        sqe = sqe + _fold(sq[4 * _Q * j:4 * _Q * (j + 1), :] * e_j)

    acc_ref[0] -= _fold(bce_f)
    acc_ref[1] += _fold(mf)
    acc_ref[2] += sqe
    acc_ref[3] += _fold(mb)
    acc_ref[4] -= _fold(bce_r)
    acc_ref[5] += _fold(mr)

    @pl.when(i == _G - 1)
    def _fin():
        s = [jnp.sum(acc_ref[k]) for k in range(6)]
        face = s[0] / s[1]
        box = s[2] / (s[3] * 4.0) * 0.5
        rig = s[4] / s[5] * 0.5
        out_ref[0, 0] = face + box + rig


def _flat_box(b):
    return b.reshape(_ROWS, _W, 4).transpose(0, 2, 1).reshape(4 * _ROWS, _W)


def kernel(gt_label, pred_label, box_target, box_pred, target_rig, pred_rig):
    gt = gt_label.reshape(_ROWS, _W)
    p = pred_label.reshape(_ROWS, _W)
    bt = _flat_box(box_target)
    bp = _flat_box(box_pred)
    tr = target_rig.reshape(_ROWS, _W)
    pr = pred_rig.reshape(_ROWS, _W)

    out = pl.pallas_call(
        _body,
        grid=(_G,),
        in_specs=[
            pl.BlockSpec((_K, _W), lambda i: (i, 0)),
            pl.BlockSpec((_K, _W), lambda i: (i, 0)),
            pl.BlockSpec((4 * _K, _W), lambda i: (i, 0)),
            pl.BlockSpec((4 * _K, _W), lambda i: (i, 0)),
            pl.BlockSpec((_K, _W), lambda i: (i, 0)),
            pl.BlockSpec((_K, _W), lambda i: (i, 0)),
        ],
        out_specs=pl.BlockSpec(memory_space=pltpu.SMEM),
        out_shape=jax.ShapeDtypeStruct((1, 1), jnp.float32),
        scratch_shapes=[
            pltpu.VMEM((6, 8, _W), jnp.float32),
            pltpu.VMEM((4 * _Q, _Q), jnp.bfloat16),
        ],
        compiler_params=pltpu.CompilerParams(
            dimension_semantics=("arbitrary",),
        ),
    )(gt, p, bt, bp, tr, pr)
    return out[0, 0]


# speedup vs baseline: 1.0059x; 1.0059x over previous
import jax
import jax.numpy as jnp
from jax.experimental import pallas as pl
from jax.experimental.pallas import tpu as pltpu

_N = 1048576
_W = 128
_ROWS = _N // _W
_K = 1024
_G = _ROWS // _K
_Q = 128


def _fold(x):
    return jnp.sum(x.reshape(-1, 8, _W), axis=0)


def _body(gt_ref, p_ref, bt_ref, bp_ref, tr_ref, pr_ref, out_ref,
          acc_ref, s_ref):
    i = pl.program_id(0)

    @pl.when(i == 0)
    def _init():
        acc_ref[...] = jnp.zeros_like(acc_ref)
        ri = jax.lax.broadcasted_iota(jnp.int32, (4 * _Q, _Q), 0)
        ki = jax.lax.broadcasted_iota(jnp.int32, (4 * _Q, _Q), 1)
        s_ref[...] = jnp.where(ri // 4 == ki, 1.0, 0.0).astype(jnp.bfloat16)

    gt = gt_ref[...]
    p = p_ref[...]
    tr = tr_ref[...]
    pr = pr_ref[...]

    one = jnp.float32(1.0)
    zero = jnp.float32(0.0)

    mask_f = gt >= 0.0
    mf = jnp.where(mask_f, one, zero)
    q = jnp.where(gt == 1.0, p, one - p)
    bce_f = mf * jnp.log(q)

    mb = jnp.abs(gt)
    mr = jnp.where(tr >= 0.0, mb, zero)
    qr = jnp.where(tr == 1.0, pr, one - pr)
    bce_r = mr * jnp.log(qr)

    d = bp_ref[...] - bt_ref[...]
    sq = d * d
    mb_bf = mb.astype(jnp.bfloat16)
    sqe = jnp.zeros((8, _W), jnp.float32)
    for j in range(_K // _Q):
        e_j = jax.lax.dot(s_ref[...], mb_bf[_Q * j:_Q * (j + 1), :],
                          preferred_element_type=jnp.float32)
        sqe = sqe + _fold(sq[4 * _Q * j:4 * _Q * (j + 1), :] * e_j)

    acc_ref[0] -= _fold(bce_f)
    acc_ref[1] += _fold(mf)
    acc_ref[2] += sqe
    acc_ref[3] += _fold(mb)
    acc_ref[4] -= _fold(bce_r)
    acc_ref[5] += _fold(mr)

    @pl.when(i == _G - 1)
    def _fin():
        s = [jnp.sum(acc_ref[k]) for k in range(6)]
        face = s[0] / s[1]
        box = s[2] / (s[3] * 4.0) * 0.5
        rig = s[4] / s[5] * 0.5
        out_ref[0, 0] = face + box + rig


def _flat_box(b):
    return b.reshape(_ROWS, _W, 4).transpose(0, 2, 1).reshape(4 * _ROWS, _W)


def kernel(gt_label, pred_label, box_target, box_pred, target_rig, pred_rig):
    gt = gt_label.reshape(_ROWS, _W)
    p = pred_label.reshape(_ROWS, _W)
    bt = _flat_box(box_target)
    bp = _flat_box(box_pred)
    tr = target_rig.reshape(_ROWS, _W)
    pr = pred_rig.reshape(_ROWS, _W)

    out = pl.pallas_call(
        _body,
        grid=(_G,),
        in_specs=[
            pl.BlockSpec((_K, _W), lambda i: (i, 0)),
            pl.BlockSpec((_K, _W), lambda i: (i, 0)),
            pl.BlockSpec((4 * _K, _W), lambda i: (i, 0)),
            pl.BlockSpec((4 * _K, _W), lambda i: (i, 0)),
            pl.BlockSpec((_K, _W), lambda i: (i, 0)),
            pl.BlockSpec((_K, _W), lambda i: (i, 0)),
        ],
        out_specs=pl.BlockSpec(memory_space=pltpu.SMEM),
        out_shape=jax.ShapeDtypeStruct((1, 1), jnp.float32),
        scratch_shapes=[
            pltpu.VMEM((6, 8, _W), jnp.float32),
            pltpu.VMEM((4 * _Q, _Q), jnp.bfloat16),
        ],
        compiler_params=pltpu.CompilerParams(
            dimension_semantics=("arbitrary",),
        ),
    )(gt, p, bt, bp, tr, pr)
    return out[0, 0]
